# baseline (speedup 1.0000x reference)
import functools

import jax
import jax.numpy as jnp
from jax.experimental import pallas as pl
from jax.experimental.pallas import tpu as pltpu


def _se_kernel(x_ref, p_ref, o_ref, *, inv_hw, c_hid):
    x = x_ref[...]
    C = x.shape[2]
    pooled = (jnp.sum(x, axis=(0, 1)) * inv_hw).reshape(1, C)
    h = jax.lax.dot_general(pooled, p_ref[0:c_hid, :],
                            (((1,), (1,)), ((), ())),
                            preferred_element_type=jnp.float32)
    h = jnp.maximum(h + p_ref[c_hid:c_hid + 1, 0:c_hid], 0.0)
    g = jnp.dot(h, p_ref[c_hid + 8:2 * c_hid + 8, :],
                preferred_element_type=jnp.float32)
    g = jax.nn.sigmoid(g + p_ref[2 * c_hid + 8:2 * c_hid + 9, :])
    o_ref[...] = x * g[0][None, None, :]


def kernel(x, w1, b1, w2, b2):
    B, C, H, W = x.shape
    Ch = w1.shape[0]
    itemsize = jnp.dtype(x.dtype).itemsize

    xt = jnp.transpose(x, (0, 2, 3, 1))

    b1_row = jnp.pad(jnp.asarray(b1, jnp.float32)[None, :], ((0, 0), (0, C - Ch)))
    params = jnp.concatenate(
        [jnp.asarray(w1, jnp.float32),
         b1_row, jnp.zeros((7, C), jnp.float32),
         jnp.asarray(w2, jnp.float32).T,
         jnp.asarray(b2, jnp.float32)[None, :],
         jnp.zeros((7, C), jnp.float32)], axis=0)
    pr = params.shape[0]

    slab_bytes = H * W * C * itemsize
    vmem_limit = int(min(64 << 20, 4 * slab_bytes + (8 << 20)))

    out_t = pl.pallas_call(
        functools.partial(_se_kernel, inv_hw=1.0 / (H * W), c_hid=Ch),
        out_shape=jax.ShapeDtypeStruct((B, H, W, C), x.dtype),
        grid=(B,),
        in_specs=[pl.BlockSpec((None, H, W, C), lambda b: (b, 0, 0, 0)),
                  pl.BlockSpec((pr, C), lambda b: (0, 0))],
        out_specs=pl.BlockSpec((None, H, W, C), lambda b: (b, 0, 0, 0)),
        compiler_params=pltpu.CompilerParams(
            dimension_semantics=("parallel",),
            vmem_limit_bytes=vmem_limit),
        cost_estimate=pl.CostEstimate(
            flops=2 * B * H * W * C + 4 * B * C * Ch,
            transcendentals=B * C,
            bytes_accessed=2 * B * C * H * W * itemsize),
    )(xt, params)
    return jnp.transpose(out_t, (0, 3, 1, 2))


# speedup vs baseline: 1.0010x; 1.0010x over previous
import functools

import jax
import jax.numpy as jnp
from jax.experimental import pallas as pl
from jax.experimental.pallas import tpu as pltpu


def _se_kernel(x_ref, w1_ref, b1_ref, w2t_ref, b2_ref, o_ref, *, inv_hw):
    x = x_ref[...]
    C = x.shape[2]
    pooled = (jnp.sum(x, axis=(0, 1)) * inv_hw).reshape(1, C)
    h = jax.lax.dot_general(pooled, w1_ref[...], (((1,), (1,)), ((), ())),
                            preferred_element_type=jnp.float32)
    h = jnp.maximum(h + b1_ref[...], 0.0)
    g = jnp.dot(h, w2t_ref[...], preferred_element_type=jnp.float32)
    g = jax.nn.sigmoid(g + b2_ref[...])
    o_ref[...] = x * g[0][None, None, :]


def kernel(x, w1, b1, w2, b2):
    B, C, H, W = x.shape
    Ch = w1.shape[0]
    itemsize = jnp.dtype(x.dtype).itemsize

    xt = jnp.transpose(x, (0, 2, 3, 1))
    w1f = jnp.asarray(w1, jnp.float32)
    b1r = jnp.asarray(b1, jnp.float32).reshape(1, Ch)
    w2t = jnp.asarray(w2, jnp.float32).T
    b2r = jnp.asarray(b2, jnp.float32).reshape(1, C)

    slab_bytes = H * W * C * itemsize
    vmem_limit = int(min(64 << 20, 4 * slab_bytes + (8 << 20)))

    out_t = pl.pallas_call(
        functools.partial(_se_kernel, inv_hw=1.0 / (H * W)),
        out_shape=jax.ShapeDtypeStruct((B, H, W, C), x.dtype),
        grid=(B,),
        in_specs=[pl.BlockSpec((None, H, W, C), lambda b: (b, 0, 0, 0)),
                  pl.BlockSpec((Ch, C), lambda b: (0, 0)),
                  pl.BlockSpec((1, Ch), lambda b: (0, 0)),
                  pl.BlockSpec((Ch, C), lambda b: (0, 0)),
                  pl.BlockSpec((1, C), lambda b: (0, 0))],
        out_specs=pl.BlockSpec((None, H, W, C), lambda b: (b, 0, 0, 0)),
        compiler_params=pltpu.CompilerParams(
            dimension_semantics=("parallel",),
            vmem_limit_bytes=vmem_limit),
        cost_estimate=pl.CostEstimate(
            flops=2 * B * H * W * C + 4 * B * C * Ch,
            transcendentals=B * C,
            bytes_accessed=2 * B * C * H * W * itemsize),
    )(xt, w1f, b1r, w2t, b2r)
    return jnp.transpose(out_t, (0, 3, 1, 2))


# speedup vs baseline: 1.0744x; 1.0733x over previous
import functools

import jax
import jax.numpy as jnp
from jax.experimental import pallas as pl
from jax.experimental.pallas import tpu as pltpu


def _se_kernel(x_ref, w1_ref, b1_ref, w2t_ref, b2_ref, o_ref, *, inv_hw):
    x = x_ref[...]
    C = x.shape[3]
    pooled = jnp.sum(x, axis=(1, 2)) * inv_hw
    h = jax.lax.dot_general(pooled, w1_ref[...], (((1,), (1,)), ((), ())),
                            preferred_element_type=jnp.float32)
    h = jnp.maximum(h + b1_ref[...], 0.0)
    g = jnp.dot(h, w2t_ref[...], preferred_element_type=jnp.float32)
    g = jax.nn.sigmoid(g + b2_ref[...])
    o_ref[...] = x * g[:, None, None, :]


def kernel(x, w1, b1, w2, b2):
    B, C, H, W = x.shape
    Ch = w1.shape[0]
    itemsize = jnp.dtype(x.dtype).itemsize

    xt = jnp.transpose(x, (0, 2, 3, 1))
    w1f = jnp.asarray(w1, jnp.float32)
    b1r = jnp.asarray(b1, jnp.float32).reshape(1, Ch)
    w2t = jnp.asarray(w2, jnp.float32).T
    b2r = jnp.asarray(b2, jnp.float32).reshape(1, C)

    bt = 2 if B % 2 == 0 else 1
    slab_bytes = bt * H * W * C * itemsize
    vmem_limit = int(min(80 << 20, 4 * slab_bytes + (8 << 20)))

    out_t = pl.pallas_call(
        functools.partial(_se_kernel, inv_hw=1.0 / (H * W)),
        out_shape=jax.ShapeDtypeStruct((B, H, W, C), x.dtype),
        grid=(B // bt,),
        in_specs=[pl.BlockSpec((bt, H, W, C), lambda b: (b, 0, 0, 0)),
                  pl.BlockSpec((Ch, C), lambda b: (0, 0)),
                  pl.BlockSpec((1, Ch), lambda b: (0, 0)),
                  pl.BlockSpec((Ch, C), lambda b: (0, 0)),
                  pl.BlockSpec((1, C), lambda b: (0, 0))],
        out_specs=pl.BlockSpec((bt, H, W, C), lambda b: (b, 0, 0, 0)),
        compiler_params=pltpu.CompilerParams(
            dimension_semantics=("parallel",),
            vmem_limit_bytes=vmem_limit),
        cost_estimate=pl.CostEstimate(
            flops=2 * B * H * W * C + 4 * B * C * Ch,
            transcendentals=B * C,
            bytes_accessed=2 * B * C * H * W * itemsize),
    )(xt, w1f, b1r, w2t, b2r)
    return jnp.transpose(out_t, (0, 3, 1, 2))


# speedup vs baseline: 1.2581x; 1.1709x over previous
import functools

import jax
import jax.numpy as jnp
from jax.experimental import pallas as pl
from jax.experimental.pallas import tpu as pltpu


def _se_kernel(x_ref, w1_ref, b1_ref, w2t_ref, b2_ref, o_ref, *, inv_hw):
    x = x_ref[...]
    C = x.shape[3]
    pooled = jnp.sum(x, axis=(1, 2)) * inv_hw
    h = jax.lax.dot_general(pooled, w1_ref[...], (((1,), (1,)), ((), ())),
                            preferred_element_type=jnp.float32)
    h = jnp.maximum(h + b1_ref[...], 0.0)
    g = jnp.dot(h, w2t_ref[...], preferred_element_type=jnp.float32)
    g = jax.nn.sigmoid(g + b2_ref[...])
    o_ref[...] = x * g[:, None, None, :]


def kernel(x, w1, b1, w2, b2):
    B, C, H, W = x.shape
    Ch = w1.shape[0]
    itemsize = jnp.dtype(x.dtype).itemsize

    xt = jnp.transpose(x, (0, 2, 3, 1))
    w1f = jnp.asarray(w1, jnp.float32)
    b1r = jnp.asarray(b1, jnp.float32).reshape(1, Ch)
    w2t = jnp.asarray(w2, jnp.float32).T
    b2r = jnp.asarray(b2, jnp.float32).reshape(1, C)

    bt = 4 if B % 4 == 0 else (2 if B % 2 == 0 else 1)
    slab_bytes = bt * H * W * C * itemsize
    vmem_limit = int(min(80 << 20, 4 * slab_bytes + (8 << 20)))

    out_t = pl.pallas_call(
        functools.partial(_se_kernel, inv_hw=1.0 / (H * W)),
        out_shape=jax.ShapeDtypeStruct((B, H, W, C), x.dtype),
        grid=(B // bt,),
        in_specs=[pl.BlockSpec((bt, H, W, C), lambda b: (b, 0, 0, 0)),
                  pl.BlockSpec((Ch, C), lambda b: (0, 0)),
                  pl.BlockSpec((1, Ch), lambda b: (0, 0)),
                  pl.BlockSpec((Ch, C), lambda b: (0, 0)),
                  pl.BlockSpec((1, C), lambda b: (0, 0))],
        out_specs=pl.BlockSpec((bt, H, W, C), lambda b: (b, 0, 0, 0)),
        compiler_params=pltpu.CompilerParams(
            dimension_semantics=("parallel",),
            vmem_limit_bytes=vmem_limit),
        cost_estimate=pl.CostEstimate(
            flops=2 * B * H * W * C + 4 * B * C * Ch,
            transcendentals=B * C,
            bytes_accessed=2 * B * C * H * W * itemsize),
    )(xt, w1f, b1r, w2t, b2r)
    return jnp.transpose(out_t, (0, 3, 1, 2))
